# baseline (speedup 1.0000x reference)
import functools

import jax
import jax.numpy as jnp
from jax import lax
from jax.experimental import pallas as pl
from jax.experimental.pallas import tpu as pltpu
from jax.experimental.pallas import tpu_sc as plsc

AUX_W = 0.01
BT = 128
NW = 32


def _route_body(n, e, g, x_ref, rw_ref, pos_ref, be_ref, aux_ref):
    xb = x_ref[...]
    rw = rw_ref[...]
    logits = lax.dot_general(xb, rw, (((1,), (1,)), ((), ())),
                             preferred_element_type=jnp.float32)
    m = jnp.max(logits, axis=1, keepdims=True)
    p = jnp.exp(logits - m)
    probs = p / jnp.sum(p, axis=1, keepdims=True)
    usage = jnp.sum(probs, axis=0, keepdims=True) / n
    aux_ref[...] = e * jnp.sum(usage * usage, axis=1, keepdims=True) * AUX_W

    col = lax.broadcasted_iota(jnp.int32, (n, e), 1)
    pm = jnp.max(probs, axis=1, keepdims=True)
    eidx = jnp.min(jnp.where(probs == pm, col, e), axis=1)
    onehot = (col == eidx[:, None]).astype(jnp.float32)

    cb = 512
    tri = (lax.broadcasted_iota(jnp.int32, (cb, cb), 0)
           >= lax.broadcasted_iota(jnp.int32, (cb, cb), 1)).astype(jnp.float32)
    counts = jnp.zeros((1, e), jnp.float32)
    ranks = []
    for c in range(n // cb):
        oh = onehot[c * cb:(c + 1) * cb]
        incl = lax.dot_general(tri, oh, (((1,), (0,)), ((), ())),
                               preferred_element_type=jnp.float32) + counts
        ranks.append(jnp.sum(incl * oh, axis=1) - 1.0)
        counts = counts + jnp.sum(oh, axis=0, keepdims=True)
    rank = jnp.concatenate(ranks, axis=0)

    pc = (((counts.astype(jnp.int32) + BT - 1) // BT) * BT).astype(jnp.float32)
    ustrict = (lax.broadcasted_iota(jnp.int32, (e, e), 0)
               < lax.broadcasted_iota(jnp.int32, (e, e), 1)).astype(jnp.float32)
    pad_off = lax.dot_general(pc, ustrict, (((1,), (0,)), ((), ())),
                              preferred_element_type=jnp.float32)
    pos = jnp.sum(onehot * pad_off, axis=1) + rank
    pos_ref[...] = pos.astype(jnp.int32)

    ends = pad_off + pc
    gstart = lax.broadcasted_iota(jnp.int32, (g, e), 0).astype(jnp.float32) * BT
    owner = jnp.sum((gstart >= ends).astype(jnp.float32), axis=1)
    total_padded = jnp.sum(pc, axis=1)
    active = (gstart[:, 0] < total_padded).astype(jnp.int32)
    be_ref[...] = jnp.concatenate(
        [jnp.minimum(owner, e - 1).astype(jnp.int32), active], axis=0)


def _ffn_body(g, be_ref, px_ref, w1_ref, w3_ref, w2_ref, y_ref):
    pid = pl.program_id(0)

    @pl.when(be_ref[g + pid] == 1)
    def _():
        xb = px_ref[...]
        h1 = lax.dot_general(xb, w1_ref[0], (((1,), (1,)), ((), ())),
                             preferred_element_type=jnp.float32)
        h3 = lax.dot_general(xb, w3_ref[0], (((1,), (1,)), ((), ())),
                             preferred_element_type=jnp.float32)
        h = h1 / (1.0 + jnp.exp(-h1)) * h3
        y_ref[...] = lax.dot_general(h, w2_ref[0], (((1,), (1,)), ((), ())),
                                     preferred_element_type=jnp.float32)


def _sc_scatter_rows(x_flat, pos2, p_rows):
    n, c = x_flat.shape
    bpw = n // NW
    ck = bpw // 2
    mesh = plsc.VectorSubcoreMesh(core_axis_name="c", subcore_axis_name="s")

    @functools.partial(
        pl.kernel, mesh=mesh,
        out_type=jax.ShapeDtypeStruct((p_rows, c), jnp.float32),
        scratch_types=[
            pltpu.VMEM((2, ck), jnp.int32),
            pltpu.VMEM((2, ck, c), jnp.float32),
            pltpu.SemaphoreType.DMA,
            pltpu.SemaphoreType.DMA,
        ],
    )
    def k(x_hbm, pos_hbm, px_hbm, idx_v, rows_v, sem_l, sem_s):
        wid = lax.axis_index("s") * 2 + lax.axis_index("c")
        base = wid * bpw
        pltpu.sync_copy(pos_hbm.at[wid], idx_v)
        l0 = pltpu.async_copy(x_hbm.at[pl.ds(base, ck)], rows_v.at[0], sem_l)
        l0.wait()
        s0 = pltpu.async_copy(rows_v.at[0], px_hbm.at[idx_v.at[0]], sem_s)
        l1 = pltpu.async_copy(x_hbm.at[pl.ds(base + ck, ck)], rows_v.at[1], sem_l)
        l1.wait()
        s0.wait()
        pltpu.async_copy(rows_v.at[1], px_hbm.at[idx_v.at[1]], sem_s).wait()

    return k(x_flat, pos2)


def _sc_gather_rows(y, pos2, n):
    _, c = y.shape
    bpw = n // NW
    ck = bpw // 2
    mesh = plsc.VectorSubcoreMesh(core_axis_name="c", subcore_axis_name="s")

    @functools.partial(
        pl.kernel, mesh=mesh,
        out_type=jax.ShapeDtypeStruct((n, c), jnp.float32),
        scratch_types=[
            pltpu.VMEM((2, ck), jnp.int32),
            pltpu.VMEM((2, ck, c), jnp.float32),
            pltpu.SemaphoreType.DMA,
            pltpu.SemaphoreType.DMA,
        ],
    )
    def k(y_hbm, pos_hbm, out_hbm, idx_v, rows_v, sem_g, sem_s):
        wid = lax.axis_index("s") * 2 + lax.axis_index("c")
        base = wid * bpw
        pltpu.sync_copy(pos_hbm.at[wid], idx_v)
        g0 = pltpu.async_copy(y_hbm.at[idx_v.at[0]], rows_v.at[0], sem_g)
        g0.wait()
        s0 = pltpu.async_copy(rows_v.at[0], out_hbm.at[pl.ds(base, ck)], sem_s)
        g1 = pltpu.async_copy(y_hbm.at[idx_v.at[1]], rows_v.at[1], sem_g)
        g1.wait()
        s0.wait()
        pltpu.async_copy(rows_v.at[1], out_hbm.at[pl.ds(base + ck, ck)], sem_s).wait()

    return k(y, pos2)


def kernel(x, router_w, w1, w2, w3):
    b, t, c = x.shape
    n = b * t
    e, h, _ = w1.shape
    g = n // BT + e
    p_rows = g * BT
    x_flat = x.reshape(n, c)

    pos, be, aux = pl.pallas_call(
        functools.partial(_route_body, n, e, g),
        out_shape=[
            jax.ShapeDtypeStruct((n,), jnp.int32),
            jax.ShapeDtypeStruct((2 * g,), jnp.int32),
            jax.ShapeDtypeStruct((1, 1), jnp.float32),
        ],
    )(x_flat, router_w)

    pos2 = pos.reshape(NW, 2, n // NW // 2)
    px = _sc_scatter_rows(x_flat, pos2, p_rows)

    def _row_idx(i, be_s):
        act = be_s[g + i]
        return (act * i + (1 - act) * (g - 1), 0)

    grid_spec = pltpu.PrefetchScalarGridSpec(
        num_scalar_prefetch=1,
        grid=(g,),
        in_specs=[
            pl.BlockSpec((BT, c), _row_idx),
            pl.BlockSpec((1, h, c), lambda i, be_s: (be_s[i], 0, 0)),
            pl.BlockSpec((1, h, c), lambda i, be_s: (be_s[i], 0, 0)),
            pl.BlockSpec((1, c, h), lambda i, be_s: (be_s[i], 0, 0)),
        ],
        out_specs=pl.BlockSpec((BT, c), _row_idx),
    )
    y = pl.pallas_call(
        functools.partial(_ffn_body, g),
        grid_spec=grid_spec,
        out_shape=jax.ShapeDtypeStruct((p_rows, c), jnp.float32),
    )(be, px, w1, w3, w2)

    out = _sc_gather_rows(y, pos2, n)
    return out.reshape(b, t, c), aux.reshape(())


# speedup vs baseline: 1.0145x; 1.0145x over previous
import functools

import jax
import jax.numpy as jnp
from jax import lax
from jax.experimental import pallas as pl
from jax.experimental.pallas import tpu as pltpu
from jax.experimental.pallas import tpu_sc as plsc

AUX_W = 0.01
BT = 128
NW = 32


def _route_body(n, e, g, x_ref, rw_ref, pos_ref, be_ref, aux_ref):
    xb = x_ref[...]
    rw = rw_ref[...]
    logits = lax.dot_general(xb, rw, (((1,), (1,)), ((), ())),
                             preferred_element_type=jnp.float32)
    m = jnp.max(logits, axis=1, keepdims=True)
    p = jnp.exp(logits - m)
    probs = p / jnp.sum(p, axis=1, keepdims=True)
    usage = jnp.sum(probs, axis=0, keepdims=True) / n
    aux_ref[...] = e * jnp.sum(usage * usage, axis=1, keepdims=True) * AUX_W

    col = lax.broadcasted_iota(jnp.int32, (n, e), 1)
    pm = jnp.max(probs, axis=1, keepdims=True)
    eidx = jnp.min(jnp.where(probs == pm, col, e), axis=1)
    onehot = (col == eidx[:, None]).astype(jnp.float32)

    cb = 512
    tri = (lax.broadcasted_iota(jnp.int32, (cb, cb), 0)
           >= lax.broadcasted_iota(jnp.int32, (cb, cb), 1)).astype(jnp.float32)
    counts = jnp.zeros((1, e), jnp.float32)
    ranks = []
    for c in range(n // cb):
        oh = onehot[c * cb:(c + 1) * cb]
        incl = lax.dot_general(tri, oh, (((1,), (0,)), ((), ())),
                               preferred_element_type=jnp.float32) + counts
        ranks.append(jnp.sum(incl * oh, axis=1) - 1.0)
        counts = counts + jnp.sum(oh, axis=0, keepdims=True)
    rank = jnp.concatenate(ranks, axis=0)

    pc = (((counts.astype(jnp.int32) + BT - 1) // BT) * BT).astype(jnp.float32)
    ustrict = (lax.broadcasted_iota(jnp.int32, (e, e), 0)
               < lax.broadcasted_iota(jnp.int32, (e, e), 1)).astype(jnp.float32)
    pad_off = lax.dot_general(pc, ustrict, (((1,), (0,)), ((), ())),
                              preferred_element_type=jnp.float32)
    pos = jnp.sum(onehot * pad_off, axis=1) + rank
    pos_ref[...] = pos.astype(jnp.int32)

    ends = pad_off + pc
    gstart = lax.broadcasted_iota(jnp.int32, (g, e), 0).astype(jnp.float32) * BT
    owner = jnp.sum((gstart >= ends).astype(jnp.float32), axis=1)
    total_padded = jnp.sum(pc, axis=1)
    active = (gstart[:, 0] < total_padded).astype(jnp.int32)
    be_ref[...] = jnp.concatenate(
        [jnp.minimum(owner, e - 1).astype(jnp.int32), active], axis=0)


def _ffn_body(g, be_ref, px_ref, w1_ref, w3_ref, w2_ref, y_ref):
    pid = pl.program_id(0)

    @pl.when(be_ref[g + pid] == 1)
    def _():
        xb = px_ref[...]
        h1 = lax.dot_general(xb, w1_ref[0], (((1,), (1,)), ((), ())),
                             preferred_element_type=jnp.float32)
        h3 = lax.dot_general(xb, w3_ref[0], (((1,), (1,)), ((), ())),
                             preferred_element_type=jnp.float32)
        h = h1 / (1.0 + jnp.exp(-h1)) * h3
        y_ref[...] = lax.dot_general(h, w2_ref[0], (((1,), (1,)), ((), ())),
                                     preferred_element_type=jnp.float32)


def _sc_scatter_rows(x_flat, pos, p_rows):
    n, c = x_flat.shape
    bpw = n // NW
    mesh = plsc.VectorSubcoreMesh(core_axis_name="c", subcore_axis_name="s")

    @functools.partial(
        pl.kernel, mesh=mesh,
        out_type=jax.ShapeDtypeStruct((p_rows, c), jnp.float32),
        scratch_types=[
            pltpu.VMEM((bpw,), jnp.int32),
            pltpu.VMEM((bpw, c), jnp.float32),
            pltpu.SemaphoreType.DMA,
        ],
    )
    def k(x_hbm, pos_hbm, px_hbm, idx_v, rows_v, sem):
        wid = lax.axis_index("s") * 2 + lax.axis_index("c")
        base = wid * bpw
        pltpu.sync_copy(pos_hbm.at[pl.ds(base, bpw)], idx_v)
        pltpu.sync_copy(x_hbm.at[pl.ds(base, bpw)], rows_v)
        pltpu.async_copy(rows_v, px_hbm.at[idx_v], sem).wait()

    return k(x_flat, pos)


def _sc_gather_rows(y, pos, n):
    _, c = y.shape
    bpw = n // NW
    mesh = plsc.VectorSubcoreMesh(core_axis_name="c", subcore_axis_name="s")

    @functools.partial(
        pl.kernel, mesh=mesh,
        out_type=jax.ShapeDtypeStruct((n, c), jnp.float32),
        scratch_types=[
            pltpu.VMEM((bpw,), jnp.int32),
            pltpu.VMEM((bpw, c), jnp.float32),
            pltpu.SemaphoreType.DMA,
        ],
    )
    def k(y_hbm, pos_hbm, out_hbm, idx_v, rows_v, sem):
        wid = lax.axis_index("s") * 2 + lax.axis_index("c")
        base = wid * bpw
        pltpu.sync_copy(pos_hbm.at[pl.ds(base, bpw)], idx_v)
        pltpu.async_copy(y_hbm.at[idx_v], rows_v, sem).wait()
        pltpu.sync_copy(rows_v, out_hbm.at[pl.ds(base, bpw)])

    return k(y, pos)


def kernel(x, router_w, w1, w2, w3):
    b, t, c = x.shape
    n = b * t
    e, h, _ = w1.shape
    g = n // BT + e
    p_rows = g * BT
    x_flat = x.reshape(n, c)

    pos, be, aux = pl.pallas_call(
        functools.partial(_route_body, n, e, g),
        out_shape=[
            jax.ShapeDtypeStruct((n,), jnp.int32),
            jax.ShapeDtypeStruct((2 * g,), jnp.int32),
            jax.ShapeDtypeStruct((1, 1), jnp.float32),
        ],
    )(x_flat, router_w)

    px = _sc_scatter_rows(x_flat, pos, p_rows)

    def _row_idx(i, be_s):
        act = be_s[g + i]
        return (act * i + (1 - act) * (g - 1), 0)

    grid_spec = pltpu.PrefetchScalarGridSpec(
        num_scalar_prefetch=1,
        grid=(g,),
        in_specs=[
            pl.BlockSpec((BT, c), _row_idx),
            pl.BlockSpec((1, h, c), lambda i, be_s: (be_s[i], 0, 0)),
            pl.BlockSpec((1, h, c), lambda i, be_s: (be_s[i], 0, 0)),
            pl.BlockSpec((1, c, h), lambda i, be_s: (be_s[i], 0, 0)),
        ],
        out_specs=pl.BlockSpec((BT, c), _row_idx),
    )
    y = pl.pallas_call(
        functools.partial(_ffn_body, g),
        grid_spec=grid_spec,
        out_shape=jax.ShapeDtypeStruct((p_rows, c), jnp.float32),
    )(be, px, w1, w3, w2)

    out = _sc_gather_rows(y, pos, n)
    return out.reshape(b, t, c), aux.reshape(())
